# speedup vs baseline: 4.1285x; 4.1285x over previous
import functools

import jax
import jax.numpy as jnp
from jax import lax
from jax.experimental import pallas as pl
from jax.experimental.pallas import tpu as pltpu

N_CNS = 2048
DV = 3
DC = 6
NUM_ITER = 20
LLR_MAX = 20.0


def _phi(x):
    x = jnp.clip(x, 8.5e-8, 16.635532)
    return jnp.log(jnp.exp(x) + 1.0) - jnp.log(jnp.exp(x) - 1.0)


def _vn_body(mv_ref, vmask_ref, llr_ref, out_ref, tot_ref):
    m = [mv_ref[j] * vmask_ref[j] for j in range(DV)]
    tot = llr_ref[...]
    for j in range(DV):
        tot = tot + m[j]
    tot_ref[...] = tot
    for j in range(DV):
        out_ref[j] = tot - m[j]


def _vn_update(mv, vmask, llr, *, v_blk=512):
    n_vns, batch = llr.shape
    grid = (n_vns // v_blk,)
    return pl.pallas_call(
        _vn_body,
        grid=grid,
        in_specs=[
            pl.BlockSpec((DV, v_blk, batch), lambda i: (0, i, 0)),
            pl.BlockSpec((DV, v_blk, 1), lambda i: (0, i, 0)),
            pl.BlockSpec((v_blk, batch), lambda i: (i, 0)),
        ],
        out_specs=[
            pl.BlockSpec((DV, v_blk, batch), lambda i: (0, i, 0)),
            pl.BlockSpec((v_blk, batch), lambda i: (i, 0)),
        ],
        out_shape=[
            jax.ShapeDtypeStruct((DV, n_vns, batch), jnp.float32),
            jax.ShapeDtypeStruct((n_vns, batch), jnp.float32),
        ],
    )(mv, vmask, llr)


def _cn_body(mc_ref, cmask_ref, out_ref):
    m = [mc_ref[k] for k in range(DC)]
    msk = [cmask_ref[k] for k in range(DC)]
    sgn = [jnp.where(msk[k] > 0.0,
                     jnp.where(m[k] < 0.0, -1.0, 1.0), 1.0) for k in range(DC)]
    mag = [jnp.where(msk[k] > 0.0,
                     _phi(jnp.clip(jnp.abs(m[k]), 0.0, LLR_MAX)), 0.0)
           for k in range(DC)]
    sign_node = sgn[0]
    mag_tot = mag[0]
    for k in range(1, DC):
        sign_node = sign_node * sgn[k]
        mag_tot = mag_tot + mag[k]
    for k in range(DC):
        out_ref[k] = (sign_node * sgn[k]) * _phi(mag_tot - mag[k])


def _cn_update(mc, cmask, *, c_blk=256):
    _, n_cns, batch = mc.shape
    grid = (n_cns // c_blk,)
    return pl.pallas_call(
        _cn_body,
        grid=grid,
        in_specs=[
            pl.BlockSpec((DC, c_blk, batch), lambda i: (0, i, 0)),
            pl.BlockSpec((DC, c_blk, 1), lambda i: (0, i, 0)),
        ],
        out_specs=pl.BlockSpec((DC, c_blk, batch), lambda i: (0, i, 0)),
        out_shape=jax.ShapeDtypeStruct((DC, n_cns, batch), jnp.float32),
    )(mc, cmask)


def _row_gather(src_flat, idx):
    return jnp.take(src_flat, idx, axis=0)


def _setup(vn_con, cn_ids, ind_cn, ind_cn_inv, n_vns):
    num_edges = vn_con.shape[0]
    e_ids = jnp.arange(num_edges, dtype=jnp.int32)

    vstart = jnp.searchsorted(vn_con, jnp.arange(n_vns, dtype=vn_con.dtype),
                              side="left").astype(jnp.int32)
    j_slot = e_ids - vstart[vn_con]
    vs = j_slot * n_vns + vn_con.astype(jnp.int32)

    cstart = jnp.searchsorted(cn_ids, jnp.arange(N_CNS, dtype=cn_ids.dtype),
                              side="left").astype(jnp.int32)
    k_slot = e_ids - cstart[cn_ids]
    cs = k_slot * N_CNS + cn_ids.astype(jnp.int32)

    n_vslots = DV * n_vns
    n_cslots = DC * N_CNS
    vs_of_p = vs[ind_cn]

    gv = jnp.zeros((n_cslots,), jnp.int32).at[cs].set(vs_of_p)
    gc = jnp.zeros((n_vslots,), jnp.int32).at[vs].set(cs[ind_cn_inv])

    cmask = jnp.zeros((n_cslots,), jnp.float32).at[cs].set(1.0)
    vmask = jnp.zeros((n_vslots,), jnp.float32).at[vs].set(1.0)
    return (gv, gc,
            cmask.reshape(DC, N_CNS, 1), vmask.reshape(DV, n_vns, 1))


def kernel(llr_ch, vn_con, cn_ids, ind_cn, ind_cn_inv):
    batch, n_vns = llr_ch.shape
    llr = -1.0 * jnp.transpose(llr_ch.astype(jnp.float32))
    gv, gc, cmask, vmask = _setup(vn_con, cn_ids, ind_cn, ind_cn_inv, n_vns)

    def body(_, mv):
        msg_v, _tot = _vn_update(mv, vmask, llr)
        mc = _row_gather(msg_v.reshape(DV * n_vns, batch), gv)
        msg_c = _cn_update(mc.reshape(DC, N_CNS, batch), cmask)
        mv_new = _row_gather(msg_c.reshape(DC * N_CNS, batch), gc)
        return mv_new.reshape(DV, n_vns, batch)

    mv0 = jnp.zeros((DV, n_vns, batch), jnp.float32)
    mv = lax.fori_loop(0, NUM_ITER, body, mv0)
    _, tot = _vn_update(mv, vmask, llr)
    return -1.0 * jnp.transpose(tot)


# speedup vs baseline: 4.1545x; 1.0063x over previous
import functools

import jax
import jax.numpy as jnp
from jax import lax
from jax.experimental import pallas as pl
from jax.experimental.pallas import tpu as pltpu
from jax.experimental.pallas import tpu_sc as plsc

N_CNS = 2048
DV = 3
DC = 6
NUM_ITER = 20
LLR_MAX = 20.0


def _phi(x):
    x = jnp.clip(x, 8.5e-8, 16.635532)
    return jnp.log(jnp.exp(x) + 1.0) - jnp.log(jnp.exp(x) - 1.0)


def _vn_body(mv_ref, vmask_ref, llr_ref, out_ref, tot_ref):
    m = [mv_ref[j] * vmask_ref[j] for j in range(DV)]
    tot = llr_ref[...]
    for j in range(DV):
        tot = tot + m[j]
    tot_ref[...] = tot
    for j in range(DV):
        out_ref[j] = tot - m[j]


def _vn_update(mv, vmask, llr, *, v_blk=512):
    n_vns, batch = llr.shape
    grid = (n_vns // v_blk,)
    return pl.pallas_call(
        _vn_body,
        grid=grid,
        in_specs=[
            pl.BlockSpec((DV, v_blk, batch), lambda i: (0, i, 0)),
            pl.BlockSpec((DV, v_blk, 1), lambda i: (0, i, 0)),
            pl.BlockSpec((v_blk, batch), lambda i: (i, 0)),
        ],
        out_specs=[
            pl.BlockSpec((DV, v_blk, batch), lambda i: (0, i, 0)),
            pl.BlockSpec((v_blk, batch), lambda i: (i, 0)),
        ],
        out_shape=[
            jax.ShapeDtypeStruct((DV, n_vns, batch), jnp.float32),
            jax.ShapeDtypeStruct((n_vns, batch), jnp.float32),
        ],
    )(mv, vmask, llr)


def _cn_body(mc_ref, cmask_ref, out_ref):
    m = [mc_ref[k] for k in range(DC)]
    msk = [cmask_ref[k] for k in range(DC)]
    sgn = [jnp.where(msk[k] > 0.0,
                     jnp.where(m[k] < 0.0, -1.0, 1.0), 1.0) for k in range(DC)]
    mag = [jnp.where(msk[k] > 0.0,
                     _phi(jnp.clip(jnp.abs(m[k]), 0.0, LLR_MAX)), 0.0)
           for k in range(DC)]
    sign_node = sgn[0]
    mag_tot = mag[0]
    for k in range(1, DC):
        sign_node = sign_node * sgn[k]
        mag_tot = mag_tot + mag[k]
    for k in range(DC):
        out_ref[k] = (sign_node * sgn[k]) * _phi(mag_tot - mag[k])


def _cn_update(mc, cmask, *, c_blk=256):
    _, n_cns, batch = mc.shape
    grid = (n_cns // c_blk,)
    return pl.pallas_call(
        _cn_body,
        grid=grid,
        in_specs=[
            pl.BlockSpec((DC, c_blk, batch), lambda i: (0, i, 0)),
            pl.BlockSpec((DC, c_blk, 1), lambda i: (0, i, 0)),
        ],
        out_specs=pl.BlockSpec((DC, c_blk, batch), lambda i: (0, i, 0)),
        out_shape=jax.ShapeDtypeStruct((DC, n_cns, batch), jnp.float32),
    )(mc, cmask)


_SC_NC = 2
_SC_NS = 16
_SC_NW = _SC_NC * _SC_NS


def _row_gather(src_flat, idx_chunks):
    nw, cpw, _ = idx_chunks.shape
    batch = src_flat.shape[1]
    mesh = plsc.VectorSubcoreMesh(core_axis_name="c", subcore_axis_name="s")

    @functools.partial(
        pl.kernel, mesh=mesh,
        out_type=jax.ShapeDtypeStruct((nw * cpw, 128, batch), jnp.float32),
        scratch_types=[
            pltpu.VMEM((cpw, 128), jnp.int32),
            pltpu.VMEM((cpw, 128, batch), jnp.float32),
            pltpu.SemaphoreType.DMA,
        ],
        compiler_params=pltpu.CompilerParams(use_tc_tiling_on_sc=False),
    )
    def gather_k(src_hbm, idx_hbm, out_hbm, idx_v, rows_v, sem):
        wid = lax.axis_index("s") * _SC_NC + lax.axis_index("c")
        pltpu.sync_copy(idx_hbm.at[wid], idx_v)
        handles = [
            pltpu.async_copy(src_hbm.at[idx_v.at[i]], rows_v.at[i], sem)
            for i in range(cpw)
        ]
        for h in handles:
            h.wait()
        pltpu.sync_copy(rows_v, out_hbm.at[pl.ds(wid * cpw, cpw)])

    return gather_k(src_flat, idx_chunks)


def _setup(vn_con, cn_ids, ind_cn, ind_cn_inv, n_vns):
    num_edges = vn_con.shape[0]
    e_ids = jnp.arange(num_edges, dtype=jnp.int32)

    vstart = jnp.searchsorted(vn_con, jnp.arange(n_vns, dtype=vn_con.dtype),
                              side="left").astype(jnp.int32)
    j_slot = e_ids - vstart[vn_con]
    vs = j_slot * n_vns + vn_con.astype(jnp.int32)

    cstart = jnp.searchsorted(cn_ids, jnp.arange(N_CNS, dtype=cn_ids.dtype),
                              side="left").astype(jnp.int32)
    k_slot = e_ids - cstart[cn_ids]
    cs = k_slot * N_CNS + cn_ids.astype(jnp.int32)

    n_vslots = DV * n_vns
    n_cslots = DC * N_CNS
    vs_of_p = vs[ind_cn]

    gv = jnp.zeros((n_cslots,), jnp.int32).at[cs].set(vs_of_p)
    gc = jnp.zeros((n_vslots,), jnp.int32).at[vs].set(cs[ind_cn_inv])

    cmask = jnp.zeros((n_cslots,), jnp.float32).at[cs].set(1.0)
    vmask = jnp.zeros((n_vslots,), jnp.float32).at[vs].set(1.0)
    return (gv, gc,
            cmask.reshape(DC, N_CNS, 1), vmask.reshape(DV, n_vns, 1))


def kernel(llr_ch, vn_con, cn_ids, ind_cn, ind_cn_inv):
    batch, n_vns = llr_ch.shape
    llr = -1.0 * jnp.transpose(llr_ch.astype(jnp.float32))
    gv, gc, cmask, vmask = _setup(vn_con, cn_ids, ind_cn, ind_cn_inv, n_vns)

    gv_chunks = gv.reshape(_SC_NW, -1, 128)
    gc_chunks = gc.reshape(_SC_NW, -1, 128)

    def body(_, mv):
        msg_v, _tot = _vn_update(mv, vmask, llr)
        mc = _row_gather(msg_v.reshape(DV * n_vns, batch), gv_chunks)
        msg_c = _cn_update(mc.reshape(DC, N_CNS, batch), cmask)
        mv_new = _row_gather(msg_c.reshape(DC * N_CNS, batch), gc_chunks)
        return mv_new.reshape(DV, n_vns, batch)

    mv0 = jnp.zeros((DV, n_vns, batch), jnp.float32)
    mv = lax.fori_loop(0, NUM_ITER, body, mv0)
    _, tot = _vn_update(mv, vmask, llr)
    return -1.0 * jnp.transpose(tot)


# speedup vs baseline: 7.6666x; 1.8454x over previous
import functools

import jax
import jax.numpy as jnp
from jax import lax
from jax.experimental import pallas as pl
from jax.experimental.pallas import tpu as pltpu
from jax.experimental.pallas import tpu_sc as plsc

N_CNS = 2048
DV = 3
DC = 6
NUM_ITER = 20
LLR_MAX = 20.0


def _phi(x):
    x = jnp.clip(x, 8.5e-8, 16.635532)
    return jnp.log(jnp.exp(x) + 1.0) - jnp.log(jnp.exp(x) - 1.0)


def _bwd_vn(msg_c_flat, gc_chunks, llr):
    n_vns, batch = llr.shape
    vpw = n_vns // _SC_NW
    spw = vpw * DV
    cpw = spw // 128
    mesh = plsc.VectorSubcoreMesh(core_axis_name="c", subcore_axis_name="s")

    @functools.partial(
        pl.kernel, mesh=mesh,
        out_type=[
            jax.ShapeDtypeStruct((DV * n_vns, batch), jnp.float32),
            jax.ShapeDtypeStruct((n_vns, batch), jnp.float32),
        ],
        scratch_types=[
            pltpu.VMEM((cpw, 128), jnp.int32),
            pltpu.VMEM((spw, batch), jnp.float32),
            pltpu.VMEM((vpw, batch), jnp.float32),
            pltpu.VMEM((spw, batch), jnp.float32),
            pltpu.VMEM((vpw, batch), jnp.float32),
            pltpu.SemaphoreType.DMA,
        ],
        compiler_params=pltpu.CompilerParams(use_tc_tiling_on_sc=False),
    )
    def bwd_vn_k(msgc_hbm, gc_hbm, llr_hbm, msgv_hbm, tot_hbm,
                 idx_v, rows_v, llr_v, out_v, tot_v, sem):
        wid = lax.axis_index("s") * _SC_NC + lax.axis_index("c")
        pltpu.sync_copy(gc_hbm.at[wid], idx_v)
        pltpu.sync_copy(llr_hbm.at[pl.ds(wid * vpw, vpw)], llr_v)
        handles = [
            pltpu.async_copy(msgc_hbm.at[idx_v.at[i]],
                             rows_v.at[pl.ds(128 * i, 128)], sem)
            for i in range(cpw)
        ]
        for h in handles:
            h.wait()

        def body(vi, carry):
            base = vi * DV
            for t in range(batch // 16):
                sl = pl.ds(16 * t, 16)
                m0 = rows_v[base, sl]
                m1 = rows_v[base + 1, sl]
                m2 = rows_v[base + 2, sl]
                tt = llr_v[vi, sl] + m0 + m1 + m2
                tot_v[vi, sl] = tt
                out_v[base, sl] = tt - m0
                out_v[base + 1, sl] = tt - m1
                out_v[base + 2, sl] = tt - m2
            return carry

        lax.fori_loop(0, vpw, body, 0)
        pltpu.sync_copy(out_v, msgv_hbm.at[pl.ds(wid * spw, spw)])
        pltpu.sync_copy(tot_v, tot_hbm.at[pl.ds(wid * vpw, vpw)])

    return bwd_vn_k(msg_c_flat, gc_chunks, llr)


def _cn_body(mc_ref, cmask_ref, out_ref):
    m = [mc_ref[k] for k in range(DC)]
    msk = [cmask_ref[k] for k in range(DC)]
    sgn = [jnp.where(msk[k] > 0.0,
                     jnp.where(m[k] < 0.0, -1.0, 1.0), 1.0) for k in range(DC)]
    mag = [jnp.where(msk[k] > 0.0,
                     _phi(jnp.clip(jnp.abs(m[k]), 0.0, LLR_MAX)), 0.0)
           for k in range(DC)]
    sign_node = sgn[0]
    mag_tot = mag[0]
    for k in range(1, DC):
        sign_node = sign_node * sgn[k]
        mag_tot = mag_tot + mag[k]
    for k in range(DC):
        out_ref[k] = (sign_node * sgn[k]) * _phi(mag_tot - mag[k])
    out_ref[DC] = jnp.zeros_like(out_ref[DC])


def _cn_update(mc, cmask, *, c_blk=256):
    _, n_cns, batch = mc.shape
    grid = (n_cns // c_blk,)
    return pl.pallas_call(
        _cn_body,
        grid=grid,
        in_specs=[
            pl.BlockSpec((DC, c_blk, batch), lambda i: (0, i, 0)),
            pl.BlockSpec((DC, c_blk, 1), lambda i: (0, i, 0)),
        ],
        out_specs=pl.BlockSpec((DC + 1, c_blk, batch), lambda i: (0, i, 0)),
        out_shape=jax.ShapeDtypeStruct((DC + 1, n_cns, batch), jnp.float32),
    )(mc, cmask)


_SC_NC = 2
_SC_NS = 16
_SC_NW = _SC_NC * _SC_NS


def _row_gather(src_flat, idx_chunks):
    nw, cpw, _ = idx_chunks.shape
    batch = src_flat.shape[1]
    mesh = plsc.VectorSubcoreMesh(core_axis_name="c", subcore_axis_name="s")

    @functools.partial(
        pl.kernel, mesh=mesh,
        out_type=jax.ShapeDtypeStruct((nw * cpw, 128, batch), jnp.float32),
        scratch_types=[
            pltpu.VMEM((cpw, 128), jnp.int32),
            pltpu.VMEM((cpw, 128, batch), jnp.float32),
            pltpu.SemaphoreType.DMA,
        ],
        compiler_params=pltpu.CompilerParams(use_tc_tiling_on_sc=False),
    )
    def gather_k(src_hbm, idx_hbm, out_hbm, idx_v, rows_v, sem):
        wid = lax.axis_index("s") * _SC_NC + lax.axis_index("c")
        pltpu.sync_copy(idx_hbm.at[wid], idx_v)
        handles = [
            pltpu.async_copy(src_hbm.at[idx_v.at[i]], rows_v.at[i], sem)
            for i in range(cpw)
        ]
        for h in handles:
            h.wait()
        pltpu.sync_copy(rows_v, out_hbm.at[pl.ds(wid * cpw, cpw)])

    return gather_k(src_flat, idx_chunks)


def _seg_slot(ids, depth):
    slot = jnp.zeros(ids.shape, jnp.int32)
    run = jnp.ones(ids.shape, jnp.bool_)
    for t in range(1, depth):
        sh = jnp.concatenate([jnp.full((t,), -1, ids.dtype), ids[:-t]])
        run = run & (ids == sh)
        slot = slot + run.astype(jnp.int32)
    return slot


def _seg_starts(ids, n_segs, num_edges):
    targets = jnp.arange(n_segs + 1, dtype=jnp.int32)
    return jnp.sum(ids.astype(jnp.int32)[None, :] < targets[:, None],
                   axis=1, dtype=jnp.int32)


def _setup(vn_con, cn_ids, ind_cn, ind_cn_inv, n_vns):
    num_edges = vn_con.shape[0]

    j_slot = _seg_slot(vn_con, DV)
    k_slot = _seg_slot(cn_ids, DC)
    vs = vn_con.astype(jnp.int32) * DV + j_slot
    cs = k_slot * N_CNS + cn_ids.astype(jnp.int32)

    vstart = _seg_starts(vn_con, n_vns, num_edges)
    cstart = _seg_starts(cn_ids, N_CNS, num_edges)
    deg_v = vstart[1:] - vstart[:-1]
    deg_c = cstart[1:] - cstart[:-1]
    vmask = (jnp.arange(DV, dtype=jnp.int32)[None, :] < deg_v[:, None])
    cmask = (jnp.arange(DC, dtype=jnp.int32)[:, None] < deg_c[None, :])

    p_of_s = jnp.minimum(cstart[:-1][None, :]
                         + jnp.arange(DC, dtype=jnp.int32)[:, None],
                         num_edges - 1)
    e_of_s = jnp.minimum(vstart[:-1][:, None]
                         + jnp.arange(DV, dtype=jnp.int32)[None, :],
                         num_edges - 1)

    gv = jnp.take(jnp.take(vs, ind_cn), p_of_s.reshape(-1))
    gc = jnp.where(vmask.reshape(-1),
                   jnp.take(jnp.take(cs, ind_cn_inv), e_of_s.reshape(-1)),
                   DC * N_CNS)
    return gv, gc, cmask.astype(jnp.float32).reshape(DC, N_CNS, 1)


def kernel(llr_ch, vn_con, cn_ids, ind_cn, ind_cn_inv):
    batch, n_vns = llr_ch.shape
    llr = -1.0 * jnp.transpose(llr_ch.astype(jnp.float32))
    gv, gc, cmask = _setup(vn_con, cn_ids, ind_cn, ind_cn_inv, n_vns)

    gv_chunks = gv.reshape(_SC_NW, -1, 128)
    gc_chunks = gc.reshape(_SC_NW, -1, 128)

    msg_c = jnp.zeros(((DC + 1) * N_CNS, batch), jnp.float32)
    for _ in range(NUM_ITER):
        msg_v, _tot = _bwd_vn(msg_c, gc_chunks, llr)
        mc = _row_gather(msg_v, gv_chunks)
        msg_c = _cn_update(mc.reshape(DC, N_CNS, batch),
                           cmask).reshape((DC + 1) * N_CNS, batch)
    _, tot = _bwd_vn(msg_c, gc_chunks, llr)
    return -1.0 * jnp.transpose(tot)


# speedup vs baseline: 9.0605x; 1.1818x over previous
import functools

import jax
import jax.numpy as jnp
from jax import lax
from jax.experimental import pallas as pl
from jax.experimental.pallas import tpu as pltpu
from jax.experimental.pallas import tpu_sc as plsc

N_CNS = 2048
DV = 3
DC = 6
NUM_ITER = 20
LLR_MAX = 20.0


def _phi(x):
    x = jnp.clip(x, 8.5e-8, 16.635532)
    return jnp.log(jnp.exp(x) + 1.0) - jnp.log(jnp.exp(x) - 1.0)


def _bwd_vn_fwd(msg_c_flat, gc_chunks, gs_chunks, llr):
    n_vns, batch = llr.shape
    vpw = n_vns // _SC_NW
    spw = vpw * DV
    cpw = spw // 128
    mesh = plsc.VectorSubcoreMesh(core_axis_name="c", subcore_axis_name="s")

    @functools.partial(
        pl.kernel, mesh=mesh,
        out_type=[
            jax.ShapeDtypeStruct((DV * n_vns, batch), jnp.float32),
            jax.ShapeDtypeStruct((n_vns, batch), jnp.float32),
        ],
        scratch_types=[
            pltpu.VMEM((cpw, 128), jnp.int32),
            pltpu.VMEM((cpw, 128), jnp.int32),
            pltpu.VMEM((spw, batch), jnp.float32),
            pltpu.VMEM((vpw, batch), jnp.float32),
            pltpu.VMEM((spw, batch), jnp.float32),
            pltpu.VMEM((vpw, batch), jnp.float32),
            pltpu.SemaphoreType.DMA,
        ],
        compiler_params=pltpu.CompilerParams(use_tc_tiling_on_sc=False),
    )
    def bwd_vn_fwd_k(msgc_hbm, gc_hbm, gs_hbm, llr_hbm, mc_hbm, tot_hbm,
                     idx_v, idx2_v, rows_v, llr_v, out_v, tot_v, sem):
        wid = lax.axis_index("s") * _SC_NC + lax.axis_index("c")
        pltpu.sync_copy(gc_hbm.at[wid], idx_v)
        pltpu.sync_copy(gs_hbm.at[wid], idx2_v)
        pltpu.sync_copy(llr_hbm.at[pl.ds(wid * vpw, vpw)], llr_v)
        handles = [
            pltpu.async_copy(msgc_hbm.at[idx_v.at[i]],
                             rows_v.at[pl.ds(128 * i, 128)], sem)
            for i in range(cpw)
        ]
        for h in handles:
            h.wait()

        def body(vi, carry):
            base = vi * DV
            for t in range(batch // 16):
                sl = pl.ds(16 * t, 16)
                m0 = rows_v[base, sl]
                m1 = rows_v[base + 1, sl]
                m2 = rows_v[base + 2, sl]
                tt = llr_v[vi, sl] + m0 + m1 + m2
                tot_v[vi, sl] = tt
                out_v[base, sl] = tt - m0
                out_v[base + 1, sl] = tt - m1
                out_v[base + 2, sl] = tt - m2
            return carry

        lax.fori_loop(0, vpw, body, 0)
        scatters = [
            pltpu.async_copy(out_v.at[pl.ds(128 * i, 128)],
                             mc_hbm.at[idx2_v.at[i]], sem)
            for i in range(cpw)
        ]
        for h in scatters:
            h.wait()
        pltpu.sync_copy(tot_v, tot_hbm.at[pl.ds(wid * vpw, vpw)])

    return bwd_vn_fwd_k(msg_c_flat, gc_chunks, gs_chunks, llr)


def _cn_body(mc_ref, cmask_ref, out_ref):
    m = [mc_ref[k] for k in range(DC)]
    msk = [cmask_ref[k] for k in range(DC)]
    sgn = [jnp.where(msk[k] > 0.0,
                     jnp.where(m[k] < 0.0, -1.0, 1.0), 1.0) for k in range(DC)]
    mag = [jnp.where(msk[k] > 0.0,
                     _phi(jnp.clip(jnp.abs(m[k]), 0.0, LLR_MAX)), 0.0)
           for k in range(DC)]
    sign_node = sgn[0]
    mag_tot = mag[0]
    for k in range(1, DC):
        sign_node = sign_node * sgn[k]
        mag_tot = mag_tot + mag[k]
    for k in range(DC):
        out_ref[k] = (sign_node * sgn[k]) * _phi(mag_tot - mag[k])
    out_ref[DC] = jnp.zeros_like(out_ref[DC])


def _cn_update(mc, cmask, *, c_blk=256):
    _, n_cns, batch = mc.shape
    grid = (n_cns // c_blk,)
    return pl.pallas_call(
        _cn_body,
        grid=grid,
        in_specs=[
            pl.BlockSpec((DC, c_blk, batch), lambda i: (0, i, 0)),
            pl.BlockSpec((DC, c_blk, 1), lambda i: (0, i, 0)),
        ],
        out_specs=pl.BlockSpec((DC + 1, c_blk, batch), lambda i: (0, i, 0)),
        out_shape=jax.ShapeDtypeStruct((DC + 1, n_cns, batch), jnp.float32),
    )(mc, cmask)


_SC_NC = 2
_SC_NS = 16
_SC_NW = _SC_NC * _SC_NS


def _row_gather(src_flat, idx_chunks):
    nw, cpw, _ = idx_chunks.shape
    batch = src_flat.shape[1]
    mesh = plsc.VectorSubcoreMesh(core_axis_name="c", subcore_axis_name="s")

    @functools.partial(
        pl.kernel, mesh=mesh,
        out_type=jax.ShapeDtypeStruct((nw * cpw, 128, batch), jnp.float32),
        scratch_types=[
            pltpu.VMEM((cpw, 128), jnp.int32),
            pltpu.VMEM((cpw, 128, batch), jnp.float32),
            pltpu.SemaphoreType.DMA,
        ],
        compiler_params=pltpu.CompilerParams(use_tc_tiling_on_sc=False),
    )
    def gather_k(src_hbm, idx_hbm, out_hbm, idx_v, rows_v, sem):
        wid = lax.axis_index("s") * _SC_NC + lax.axis_index("c")
        pltpu.sync_copy(idx_hbm.at[wid], idx_v)
        handles = [
            pltpu.async_copy(src_hbm.at[idx_v.at[i]], rows_v.at[i], sem)
            for i in range(cpw)
        ]
        for h in handles:
            h.wait()
        pltpu.sync_copy(rows_v, out_hbm.at[pl.ds(wid * cpw, cpw)])

    return gather_k(src_flat, idx_chunks)


def _seg_slot(ids, depth):
    slot = jnp.zeros(ids.shape, jnp.int32)
    run = jnp.ones(ids.shape, jnp.bool_)
    for t in range(1, depth):
        sh = jnp.concatenate([jnp.full((t,), -1, ids.dtype), ids[:-t]])
        run = run & (ids == sh)
        slot = slot + run.astype(jnp.int32)
    return slot


def _seg_starts(ids, n_segs, num_edges):
    targets = jnp.arange(n_segs + 1, dtype=jnp.int32)
    return jnp.sum(ids.astype(jnp.int32)[None, :] < targets[:, None],
                   axis=1, dtype=jnp.int32)


def _setup(vn_con, cn_ids, ind_cn, ind_cn_inv, n_vns):
    num_edges = vn_con.shape[0]

    j_slot = _seg_slot(vn_con, DV)
    k_slot = _seg_slot(cn_ids, DC)
    cs = k_slot * N_CNS + cn_ids.astype(jnp.int32)

    vstart = _seg_starts(vn_con, n_vns, num_edges)
    cstart = _seg_starts(cn_ids, N_CNS, num_edges)
    deg_v = vstart[1:] - vstart[:-1]
    deg_c = cstart[1:] - cstart[:-1]
    vmask = (jnp.arange(DV, dtype=jnp.int32)[None, :] < deg_v[:, None])
    cmask = (jnp.arange(DC, dtype=jnp.int32)[:, None] < deg_c[None, :])

    e_of_s = jnp.minimum(vstart[:-1][:, None]
                         + jnp.arange(DV, dtype=jnp.int32)[None, :],
                         num_edges - 1)

    base = jnp.take(jnp.take(cs, ind_cn_inv), e_of_s.reshape(-1))
    vmask_flat = vmask.reshape(-1)
    cmask_f = cmask.astype(jnp.float32)
    dummy = jnp.argmin(cmask_f.reshape(-1)).astype(jnp.int32)
    gc = jnp.where(vmask_flat, base, DC * N_CNS)
    gs = jnp.where(vmask_flat, base, dummy)
    return gc, gs, cmask_f.reshape(DC, N_CNS, 1)


def kernel(llr_ch, vn_con, cn_ids, ind_cn, ind_cn_inv):
    batch, n_vns = llr_ch.shape
    llr = -1.0 * jnp.transpose(llr_ch.astype(jnp.float32))
    gc, gs, cmask = _setup(vn_con, cn_ids, ind_cn, ind_cn_inv, n_vns)

    gc_chunks = gc.reshape(_SC_NW, -1, 128)
    gs_chunks = gs.reshape(_SC_NW, -1, 128)

    msg_c = jnp.zeros(((DC + 1) * N_CNS, batch), jnp.float32)
    for _ in range(NUM_ITER):
        mc, _tot = _bwd_vn_fwd(msg_c, gc_chunks, gs_chunks, llr)
        msg_c = _cn_update(mc.reshape(DC, N_CNS, batch),
                           cmask).reshape((DC + 1) * N_CNS, batch)
    _, tot = _bwd_vn_fwd(msg_c, gc_chunks, gs_chunks, llr)
    return -1.0 * jnp.transpose(tot)


# speedup vs baseline: 9.0877x; 1.0030x over previous
import functools

import jax
import jax.numpy as jnp
from jax import lax
from jax.experimental import pallas as pl
from jax.experimental.pallas import tpu as pltpu
from jax.experimental.pallas import tpu_sc as plsc

N_CNS = 2048
DV = 3
DC = 6
NUM_ITER = 20
LLR_MAX = 20.0


def _phi(x):
    x = jnp.clip(x, 8.5e-8, 16.635532)
    t = jnp.exp(x)
    return jnp.log((t + 1.0) / (t - 1.0))


def _bwd_vn_fwd(msg_c_flat, gc_chunks, gs_chunks, llr):
    n_vns, batch = llr.shape
    vpw = n_vns // _SC_NW
    spw = vpw * DV
    cpw = spw // 128
    mesh = plsc.VectorSubcoreMesh(core_axis_name="c", subcore_axis_name="s")

    @functools.partial(
        pl.kernel, mesh=mesh,
        out_type=[
            jax.ShapeDtypeStruct((DV * n_vns, batch), jnp.float32),
            jax.ShapeDtypeStruct((n_vns, batch), jnp.float32),
        ],
        scratch_types=[
            pltpu.VMEM((cpw, 128), jnp.int32),
            pltpu.VMEM((cpw, 128), jnp.int32),
            pltpu.VMEM((spw, batch), jnp.float32),
            pltpu.VMEM((vpw, batch), jnp.float32),
            pltpu.VMEM((spw, batch), jnp.float32),
            pltpu.VMEM((vpw, batch), jnp.float32),
            pltpu.SemaphoreType.DMA,
        ],
        compiler_params=pltpu.CompilerParams(use_tc_tiling_on_sc=False),
    )
    def bwd_vn_fwd_k(msgc_hbm, gc_hbm, gs_hbm, llr_hbm, mc_hbm, tot_hbm,
                     idx_v, idx2_v, rows_v, llr_v, out_v, tot_v, sem):
        wid = lax.axis_index("s") * _SC_NC + lax.axis_index("c")
        pltpu.sync_copy(gc_hbm.at[wid], idx_v)
        pltpu.sync_copy(gs_hbm.at[wid], idx2_v)
        pltpu.sync_copy(llr_hbm.at[pl.ds(wid * vpw, vpw)], llr_v)
        handles = [
            pltpu.async_copy(msgc_hbm.at[idx_v.at[i]],
                             rows_v.at[pl.ds(128 * i, 128)], sem)
            for i in range(cpw)
        ]
        for h in handles:
            h.wait()

        def body(vi, carry):
            base = vi * DV
            for t in range(batch // 16):
                sl = pl.ds(16 * t, 16)
                m0 = rows_v[base, sl]
                m1 = rows_v[base + 1, sl]
                m2 = rows_v[base + 2, sl]
                tt = llr_v[vi, sl] + m0 + m1 + m2
                tot_v[vi, sl] = tt
                out_v[base, sl] = tt - m0
                out_v[base + 1, sl] = tt - m1
                out_v[base + 2, sl] = tt - m2
            return carry

        lax.fori_loop(0, vpw, body, 0)
        scatters = [
            pltpu.async_copy(out_v.at[pl.ds(128 * i, 128)],
                             mc_hbm.at[idx2_v.at[i]], sem)
            for i in range(cpw)
        ]
        for h in scatters:
            h.wait()
        pltpu.sync_copy(tot_v, tot_hbm.at[pl.ds(wid * vpw, vpw)])

    return bwd_vn_fwd_k(msg_c_flat, gc_chunks, gs_chunks, llr)


def _cn_body(mc_ref, cmask_ref, out_ref):
    m = [mc_ref[k] for k in range(DC)]
    msk = [cmask_ref[k] for k in range(DC)]
    sgn = [jnp.where(msk[k] > 0.0,
                     jnp.where(m[k] < 0.0, -1.0, 1.0), 1.0) for k in range(DC)]
    mag = [jnp.where(msk[k] > 0.0,
                     _phi(jnp.clip(jnp.abs(m[k]), 0.0, LLR_MAX)), 0.0)
           for k in range(DC)]
    sign_node = sgn[0]
    mag_tot = mag[0]
    for k in range(1, DC):
        sign_node = sign_node * sgn[k]
        mag_tot = mag_tot + mag[k]
    for k in range(DC):
        out_ref[k] = (sign_node * sgn[k]) * _phi(mag_tot - mag[k])
    out_ref[DC] = jnp.zeros_like(out_ref[DC])


def _cn_update(mc, cmask, *, c_blk=256):
    _, n_cns, batch = mc.shape
    grid = (n_cns // c_blk,)
    return pl.pallas_call(
        _cn_body,
        grid=grid,
        in_specs=[
            pl.BlockSpec((DC, c_blk, batch), lambda i: (0, i, 0)),
            pl.BlockSpec((DC, c_blk, 1), lambda i: (0, i, 0)),
        ],
        out_specs=pl.BlockSpec((DC + 1, c_blk, batch), lambda i: (0, i, 0)),
        out_shape=jax.ShapeDtypeStruct((DC + 1, n_cns, batch), jnp.float32),
    )(mc, cmask)


_SC_NC = 2
_SC_NS = 16
_SC_NW = _SC_NC * _SC_NS


def _row_gather(src_flat, idx_chunks):
    nw, cpw, _ = idx_chunks.shape
    batch = src_flat.shape[1]
    mesh = plsc.VectorSubcoreMesh(core_axis_name="c", subcore_axis_name="s")

    @functools.partial(
        pl.kernel, mesh=mesh,
        out_type=jax.ShapeDtypeStruct((nw * cpw, 128, batch), jnp.float32),
        scratch_types=[
            pltpu.VMEM((cpw, 128), jnp.int32),
            pltpu.VMEM((cpw, 128, batch), jnp.float32),
            pltpu.SemaphoreType.DMA,
        ],
        compiler_params=pltpu.CompilerParams(use_tc_tiling_on_sc=False),
    )
    def gather_k(src_hbm, idx_hbm, out_hbm, idx_v, rows_v, sem):
        wid = lax.axis_index("s") * _SC_NC + lax.axis_index("c")
        pltpu.sync_copy(idx_hbm.at[wid], idx_v)
        handles = [
            pltpu.async_copy(src_hbm.at[idx_v.at[i]], rows_v.at[i], sem)
            for i in range(cpw)
        ]
        for h in handles:
            h.wait()
        pltpu.sync_copy(rows_v, out_hbm.at[pl.ds(wid * cpw, cpw)])

    return gather_k(src_flat, idx_chunks)


def _seg_slot(ids, depth):
    slot = jnp.zeros(ids.shape, jnp.int32)
    run = jnp.ones(ids.shape, jnp.bool_)
    for t in range(1, depth):
        sh = jnp.concatenate([jnp.full((t,), -1, ids.dtype), ids[:-t]])
        run = run & (ids == sh)
        slot = slot + run.astype(jnp.int32)
    return slot


def _seg_starts(ids, n_segs, num_edges):
    targets = jnp.arange(n_segs + 1, dtype=jnp.int32)
    return jnp.sum(ids.astype(jnp.int32)[None, :] < targets[:, None],
                   axis=1, dtype=jnp.int32)


def _setup(vn_con, cn_ids, ind_cn, ind_cn_inv, n_vns):
    num_edges = vn_con.shape[0]

    j_slot = _seg_slot(vn_con, DV)
    k_slot = _seg_slot(cn_ids, DC)
    cs = k_slot * N_CNS + cn_ids.astype(jnp.int32)

    vstart = _seg_starts(vn_con, n_vns, num_edges)
    cstart = _seg_starts(cn_ids, N_CNS, num_edges)
    deg_v = vstart[1:] - vstart[:-1]
    deg_c = cstart[1:] - cstart[:-1]
    vmask = (jnp.arange(DV, dtype=jnp.int32)[None, :] < deg_v[:, None])
    cmask = (jnp.arange(DC, dtype=jnp.int32)[:, None] < deg_c[None, :])

    e_of_s = jnp.minimum(vstart[:-1][:, None]
                         + jnp.arange(DV, dtype=jnp.int32)[None, :],
                         num_edges - 1)

    base = jnp.take(jnp.take(cs, ind_cn_inv), e_of_s.reshape(-1))
    vmask_flat = vmask.reshape(-1)
    cmask_f = cmask.astype(jnp.float32)
    dummy = jnp.argmin(cmask_f.reshape(-1)).astype(jnp.int32)
    gc = jnp.where(vmask_flat, base, DC * N_CNS)
    gs = jnp.where(vmask_flat, base, dummy)
    return gc, gs, cmask_f.reshape(DC, N_CNS, 1)


def kernel(llr_ch, vn_con, cn_ids, ind_cn, ind_cn_inv):
    batch, n_vns = llr_ch.shape
    llr = -1.0 * jnp.transpose(llr_ch.astype(jnp.float32))
    gc, gs, cmask = _setup(vn_con, cn_ids, ind_cn, ind_cn_inv, n_vns)

    gc_chunks = gc.reshape(_SC_NW, -1, 128)
    gs_chunks = gs.reshape(_SC_NW, -1, 128)

    msg_c = jnp.zeros(((DC + 1) * N_CNS, batch), jnp.float32)
    for _ in range(NUM_ITER):
        mc, _tot = _bwd_vn_fwd(msg_c, gc_chunks, gs_chunks, llr)
        msg_c = _cn_update(mc.reshape(DC, N_CNS, batch),
                           cmask).reshape((DC + 1) * N_CNS, batch)
    _, tot = _bwd_vn_fwd(msg_c, gc_chunks, gs_chunks, llr)
    return -1.0 * jnp.transpose(tot)
